# baseline (speedup 1.0000x reference)
import functools

import jax
import jax.numpy as jnp
from jax import lax
from jax.experimental import pallas as pl
from jax.experimental.pallas import tpu as pltpu
from jax.experimental.pallas import tpu_sc as plsc

_B, _S, _H = 4, 8192, 2048
_AD, _R, _K = 32, 16, 8
_TEMP = 0.1

_S_TC = 6144
_S_SC = _S - _S_TC
_S_CHUNK = 128
_NW = 32
_WPB = _NW // _B
_RPW = _S_SC // _WPB
_C = 16
_N_CHUNKS = _RPW // _C
_N_PAIRS = _N_CHUNKS // 2
_CT = 512
_N_CT = _H // _CT
_LANES = 16


def _sc_reduce_kernel(hs_ref, out_ref, buf0, buf1, acc_ref, sem0, sem1):
    c = lax.axis_index("c")
    s = lax.axis_index("s")
    wid = s * 2 + c
    b = wid // _WPB
    j = wid % _WPB
    base = b * _S + _S_TC + j * _RPW

    zero16 = jnp.zeros((_LANES,), jnp.float32)
    for v in range(_H // _LANES):
        acc_ref[pl.ds(v * _LANES, _LANES)] = zero16

    def start(chunk, buf, sem):
        pltpu.async_copy(hs_ref.at[pl.ds(base + chunk * _C, _C)], buf, sem)

    def wait(buf, sem):
        pltpu.make_async_copy(hs_ref.at[pl.ds(0, _C)], buf, sem).wait()

    def process(buf):
        for ct in range(_N_CT):
            col0 = ct * _CT
            accs = tuple(acc_ref[pl.ds(col0 + v * _LANES, _LANES)]
                         for v in range(_CT // _LANES))

            def rbody(r, accs):
                return tuple(
                    a + buf[r, pl.ds(col0 + v * _LANES, _LANES)]
                    for v, a in enumerate(accs))

            accs = lax.fori_loop(0, _C, rbody, accs)
            for v in range(_CT // _LANES):
                acc_ref[pl.ds(col0 + v * _LANES, _LANES)] = accs[v]

    start(0, buf0, sem0)
    start(1, buf1, sem1)

    def pbody(p, carry):
        wait(buf0, sem0)
        process(buf0)
        start(2 * p + 2, buf0, sem0)
        wait(buf1, sem1)
        process(buf1)
        start(2 * p + 3, buf1, sem1)
        return carry

    lax.fori_loop(0, _N_PAIRS - 1, pbody, 0)
    wait(buf0, sem0)
    process(buf0)
    wait(buf1, sem1)
    process(buf1)

    pltpu.sync_copy(acc_ref, out_ref.at[wid])


def _sc_reduce(hs2):
    mesh = plsc.VectorSubcoreMesh(core_axis_name="c", subcore_axis_name="s")
    return pl.kernel(
        _sc_reduce_kernel,
        out_type=jax.ShapeDtypeStruct((_NW, _H), jnp.float32),
        mesh=mesh,
        scratch_types=[
            pltpu.VMEM((_C, _H), jnp.float32),
            pltpu.VMEM((_C, _H), jnp.float32),
            pltpu.VMEM((_H,), jnp.float32),
            pltpu.SemaphoreType.DMA,
            pltpu.SemaphoreType.DMA,
        ],
    )(hs2)


def _tc_reduce_kernel(hs_ref, out_ref):
    @pl.when(pl.program_id(0) == 0)
    def _():
        out_ref[...] = jnp.zeros_like(out_ref)

    out_ref[...] += jnp.sum(hs_ref[...], axis=1)


def _tc_reduce(hidden_states):
    return pl.pallas_call(
        _tc_reduce_kernel,
        grid=(_S_TC // _S_CHUNK,),
        in_specs=[pl.BlockSpec((_B, _S_CHUNK, _H), lambda i: (0, i, 0))],
        out_specs=pl.BlockSpec((_B, _H), lambda i: (0, 0)),
        out_shape=jax.ShapeDtypeStruct((_B, _H), jnp.float32),
    )(hidden_states)


def _tail_kernel(tcp_ref, scp_ref, W1_ref, b1_ref, gamma_ref, beta_ref,
                 W2_ref, b2_ref, ml_ref, keep_ref, gumbel_ref, train_ref,
                 out_ref):
    sc_sum = jnp.sum(scp_ref[...].reshape(_B, _WPB, _H), axis=1)
    pooled = (tcp_ref[...] + sc_sum) * (1.0 / _S)

    x = jnp.dot(pooled, W1_ref[...], preferred_element_type=jnp.float32)
    x = x + b1_ref[...]
    mu = jnp.mean(x, axis=-1, keepdims=True)
    var = jnp.mean((x - mu) ** 2, axis=-1, keepdims=True)
    x = (x - mu) / jnp.sqrt(var + 1e-5) * gamma_ref[...] + beta_ref[...]
    x = 0.5 * x * (1.0 + lax.erf(x / jnp.sqrt(2.0).astype(jnp.float32)))
    x_dropped = jnp.where(keep_ref[...] > 0.5, x / 0.9, 0.0)
    is_training = train_ref[0, 0] != 0
    x = jnp.where(is_training, x_dropped, x)
    logits = (jnp.dot(x, W2_ref[...], preferred_element_type=jnp.float32)
              + b2_ref[...] + ml_ref[...])

    def khot(v):
        col = lax.broadcasted_iota(jnp.int32, v.shape, 1)
        cnt = jnp.zeros(v.shape, jnp.int32)
        for k in range(_R):
            vk = v[:, k][:, None]
            beats = (vk > v) | ((vk == v) & (k < col))
            cnt = cnt + beats.astype(jnp.int32)
        return (cnt < _K).astype(jnp.float32)

    noisy = (logits + gumbel_ref[...]) / _TEMP
    hard = khot(noisy)
    z = logits / _TEMP
    z = z - jnp.max(z, axis=-1, keepdims=True)
    ez = jnp.exp(z)
    soft = ez / jnp.sum(ez, axis=-1, keepdims=True)
    mask_train = hard + soft - soft
    mask_eval = khot(logits)
    out_ref[...] = jnp.where(is_training, mask_train, mask_eval)


def _tail(tc_part, sc_part, W1, b1, gamma, beta, W2, b2, mask_logits,
          keep, gumbel, train_arr):
    whole = lambda r, c: pl.BlockSpec((r, c), lambda: (0, 0))
    return pl.pallas_call(
        _tail_kernel,
        in_specs=[
            whole(_B, _H),
            whole(_NW, _H),
            whole(_H, _AD),
            whole(1, _AD),
            whole(1, _AD),
            whole(1, _AD),
            whole(_AD, _R),
            whole(1, _R),
            whole(1, _R),
            whole(_B, _AD),
            whole(_B, _R),
            whole(1, 1),
        ],
        out_specs=whole(_B, _R),
        out_shape=jax.ShapeDtypeStruct((_B, _R), jnp.float32),
    )(tc_part, sc_part, W1, b1.reshape(1, _AD), gamma.reshape(1, _AD),
      beta.reshape(1, _AD), W2, b2.reshape(1, _R),
      mask_logits.reshape(1, _R), keep, gumbel, train_arr)


def kernel(hidden_states, W1, b1, gamma, beta, W2, b2, mask_logits,
           training):
    keep = jax.random.bernoulli(jax.random.key(42), 0.9,
                                (_B, _AD)).astype(jnp.float32)
    u = jax.random.uniform(jax.random.key(7), (_B, _R), dtype=jnp.float32)
    gumbel = -jnp.log(-jnp.log(u + 1e-8) + 1e-8)
    train_arr = jnp.asarray(training, jnp.float32).reshape(1, 1)

    hs2 = hidden_states.reshape(_B * _S, _H)
    sc_part = _sc_reduce(hs2)
    tc_part = _tc_reduce(hidden_states)
    return _tail(tc_part, sc_part, W1, b1, gamma, beta, W2, b2,
                 mask_logits, keep, gumbel, train_arr)


# speedup vs baseline: 1.0005x; 1.0005x over previous
import functools

import jax
import jax.numpy as jnp
from jax import lax
from jax.experimental import pallas as pl
from jax.experimental.pallas import tpu as pltpu
from jax.experimental.pallas import tpu_sc as plsc

_B, _S, _H = 4, 8192, 2048
_AD, _R, _K = 32, 16, 8
_TEMP = 0.1

_S_TC = 5120
_S_SC = _S - _S_TC
_S_CHUNK = 128
_NW = 32
_WPB = _NW // _B
_RPW = _S_SC // _WPB
_C = 16
_N_CHUNKS = _RPW // _C
_N_PAIRS = _N_CHUNKS // 2
_CT = 512
_N_CT = _H // _CT
_LANES = 16


def _sc_reduce_kernel(hs_ref, out_ref, buf0, buf1, acc_ref, sem0, sem1):
    c = lax.axis_index("c")
    s = lax.axis_index("s")
    wid = s * 2 + c
    b = wid // _WPB
    j = wid % _WPB
    base = b * _S + _S_TC + j * _RPW

    zero16 = jnp.zeros((_LANES,), jnp.float32)
    for v in range(_H // _LANES):
        acc_ref[pl.ds(v * _LANES, _LANES)] = zero16

    def start(chunk, buf, sem):
        pltpu.async_copy(hs_ref.at[pl.ds(base + chunk * _C, _C)], buf, sem)

    def wait(buf, sem):
        pltpu.make_async_copy(hs_ref.at[pl.ds(0, _C)], buf, sem).wait()

    def process(buf):
        for ct in range(_N_CT):
            col0 = ct * _CT
            accs = tuple(acc_ref[pl.ds(col0 + v * _LANES, _LANES)]
                         for v in range(_CT // _LANES))

            def rbody(r, accs):
                return tuple(
                    a + buf[r, pl.ds(col0 + v * _LANES, _LANES)]
                    for v, a in enumerate(accs))

            accs = lax.fori_loop(0, _C, rbody, accs)
            for v in range(_CT // _LANES):
                acc_ref[pl.ds(col0 + v * _LANES, _LANES)] = accs[v]

    start(0, buf0, sem0)
    start(1, buf1, sem1)

    def pbody(p, carry):
        wait(buf0, sem0)
        process(buf0)
        start(2 * p + 2, buf0, sem0)
        wait(buf1, sem1)
        process(buf1)
        start(2 * p + 3, buf1, sem1)
        return carry

    lax.fori_loop(0, _N_PAIRS - 1, pbody, 0)
    wait(buf0, sem0)
    process(buf0)
    wait(buf1, sem1)
    process(buf1)

    pltpu.sync_copy(acc_ref, out_ref.at[wid])


def _sc_reduce(hs2):
    mesh = plsc.VectorSubcoreMesh(core_axis_name="c", subcore_axis_name="s")
    return pl.kernel(
        _sc_reduce_kernel,
        out_type=jax.ShapeDtypeStruct((_NW, _H), jnp.float32),
        mesh=mesh,
        scratch_types=[
            pltpu.VMEM((_C, _H), jnp.float32),
            pltpu.VMEM((_C, _H), jnp.float32),
            pltpu.VMEM((_H,), jnp.float32),
            pltpu.SemaphoreType.DMA,
            pltpu.SemaphoreType.DMA,
        ],
    )(hs2)


def _tc_reduce_kernel(hs_ref, out_ref):
    @pl.when(pl.program_id(0) == 0)
    def _():
        out_ref[...] = jnp.zeros_like(out_ref)

    out_ref[...] += jnp.sum(hs_ref[...], axis=1)


def _tc_reduce(hidden_states):
    return pl.pallas_call(
        _tc_reduce_kernel,
        grid=(_S_TC // _S_CHUNK,),
        in_specs=[pl.BlockSpec((_B, _S_CHUNK, _H), lambda i: (0, i, 0))],
        out_specs=pl.BlockSpec((_B, _H), lambda i: (0, 0)),
        out_shape=jax.ShapeDtypeStruct((_B, _H), jnp.float32),
    )(hidden_states)


def _tail_kernel(tcp_ref, scp_ref, W1_ref, b1_ref, gamma_ref, beta_ref,
                 W2_ref, b2_ref, ml_ref, keep_ref, gumbel_ref, train_ref,
                 out_ref):
    sc_sum = jnp.sum(scp_ref[...].reshape(_B, _WPB, _H), axis=1)
    pooled = (tcp_ref[...] + sc_sum) * (1.0 / _S)

    x = jnp.dot(pooled, W1_ref[...], preferred_element_type=jnp.float32)
    x = x + b1_ref[...]
    mu = jnp.mean(x, axis=-1, keepdims=True)
    var = jnp.mean((x - mu) ** 2, axis=-1, keepdims=True)
    x = (x - mu) / jnp.sqrt(var + 1e-5) * gamma_ref[...] + beta_ref[...]
    x = 0.5 * x * (1.0 + lax.erf(x / jnp.sqrt(2.0).astype(jnp.float32)))
    x_dropped = jnp.where(keep_ref[...] > 0.5, x / 0.9, 0.0)
    is_training = train_ref[0, 0] != 0
    x = jnp.where(is_training, x_dropped, x)
    logits = (jnp.dot(x, W2_ref[...], preferred_element_type=jnp.float32)
              + b2_ref[...] + ml_ref[...])

    def khot(v):
        col = lax.broadcasted_iota(jnp.int32, v.shape, 1)
        cnt = jnp.zeros(v.shape, jnp.int32)
        for k in range(_R):
            vk = v[:, k][:, None]
            beats = (vk > v) | ((vk == v) & (k < col))
            cnt = cnt + beats.astype(jnp.int32)
        return (cnt < _K).astype(jnp.float32)

    noisy = (logits + gumbel_ref[...]) / _TEMP
    hard = khot(noisy)
    z = logits / _TEMP
    z = z - jnp.max(z, axis=-1, keepdims=True)
    ez = jnp.exp(z)
    soft = ez / jnp.sum(ez, axis=-1, keepdims=True)
    mask_train = hard + soft - soft
    mask_eval = khot(logits)
    out_ref[...] = jnp.where(is_training, mask_train, mask_eval)


def _tail(tc_part, sc_part, W1, b1, gamma, beta, W2, b2, mask_logits,
          keep, gumbel, train_arr):
    whole = lambda r, c: pl.BlockSpec((r, c), lambda: (0, 0))
    return pl.pallas_call(
        _tail_kernel,
        in_specs=[
            whole(_B, _H),
            whole(_NW, _H),
            whole(_H, _AD),
            whole(1, _AD),
            whole(1, _AD),
            whole(1, _AD),
            whole(_AD, _R),
            whole(1, _R),
            whole(1, _R),
            whole(_B, _AD),
            whole(_B, _R),
            whole(1, 1),
        ],
        out_specs=whole(_B, _R),
        out_shape=jax.ShapeDtypeStruct((_B, _R), jnp.float32),
    )(tc_part, sc_part, W1, b1.reshape(1, _AD), gamma.reshape(1, _AD),
      beta.reshape(1, _AD), W2, b2.reshape(1, _R),
      mask_logits.reshape(1, _R), keep, gumbel, train_arr)


def kernel(hidden_states, W1, b1, gamma, beta, W2, b2, mask_logits,
           training):
    keep = jax.random.bernoulli(jax.random.key(42), 0.9,
                                (_B, _AD)).astype(jnp.float32)
    u = jax.random.uniform(jax.random.key(7), (_B, _R), dtype=jnp.float32)
    gumbel = -jnp.log(-jnp.log(u + 1e-8) + 1e-8)
    train_arr = jnp.asarray(training, jnp.float32).reshape(1, 1)

    hs2 = hidden_states.reshape(_B * _S, _H)
    sc_part = _sc_reduce(hs2)
    tc_part = _tc_reduce(hidden_states)
    return _tail(tc_part, sc_part, W1, b1, gamma, beta, W2, b2,
                 mask_logits, keep, gumbel, train_arr)


# speedup vs baseline: 1.2312x; 1.2306x over previous
import functools

import jax
import jax.numpy as jnp
from jax import lax
from jax.experimental import pallas as pl
from jax.experimental.pallas import tpu as pltpu

_B, _S, _H = 4, 8192, 2048
_AD, _R, _K = 32, 16, 8
_TEMP = 0.1

_RC = 512
_CPB = _S // _RC
_N_STEPS = (_B * _S) // _RC


def _tail(pooled, W1, b1, gamma, beta, W2, b2, mask_logits, keep, gumbel,
          training):
    x = jnp.dot(pooled, W1, preferred_element_type=jnp.float32) + b1
    mu = jnp.mean(x, axis=-1, keepdims=True)
    var = jnp.mean((x - mu) ** 2, axis=-1, keepdims=True)
    x = (x - mu) / jnp.sqrt(var + 1e-5) * gamma + beta
    x = 0.5 * x * (1.0 + lax.erf(x / jnp.sqrt(2.0).astype(jnp.float32)))
    x_dropped = jnp.where(keep > 0.5, x / 0.9, 0.0)
    is_training = training != 0
    x = jnp.where(is_training, x_dropped, x)
    logits = (jnp.dot(x, W2, preferred_element_type=jnp.float32) + b2
              + mask_logits)

    def khot(v):
        col = lax.broadcasted_iota(jnp.int32, v.shape, 1)
        cnt = jnp.zeros(v.shape, jnp.int32)
        for k in range(_R):
            vk = v[:, k][:, None]
            beats = (vk > v) | ((vk == v) & (k < col))
            cnt = cnt + beats.astype(jnp.int32)
        return (cnt < _K).astype(jnp.float32)

    noisy = (logits + gumbel) / _TEMP
    hard = khot(noisy)
    z = logits / _TEMP
    z = z - jnp.max(z, axis=-1, keepdims=True)
    ez = jnp.exp(z)
    soft = ez / jnp.sum(ez, axis=-1, keepdims=True)
    mask_train = hard + soft - soft
    mask_eval = khot(logits)
    return jnp.where(is_training, mask_train, mask_eval)


def _fused_kernel(hs_ref, W1_ref, b1_ref, gamma_ref, beta_ref, W2_ref,
                  b2_ref, ml_ref, keep_ref, gumbel_ref, train_ref,
                  out_ref, acc_ref):
    c = pl.program_id(0)
    bb = c // _CPB
    part = jnp.sum(hs_ref[...], axis=0, keepdims=True)

    @pl.when(c % _CPB == 0)
    def _():
        acc_ref[pl.ds(bb, 1), :] = part

    @pl.when(c % _CPB != 0)
    def _():
        acc_ref[pl.ds(bb, 1), :] += part

    @pl.when(c == _N_STEPS - 1)
    def _():
        pooled = acc_ref[...] * (1.0 / _S)
        out_ref[...] = _tail(
            pooled, W1_ref[...], b1_ref[...], gamma_ref[...], beta_ref[...],
            W2_ref[...], b2_ref[...], ml_ref[...], keep_ref[...],
            gumbel_ref[...], train_ref[0, 0])


def kernel(hidden_states, W1, b1, gamma, beta, W2, b2, mask_logits,
           training):
    keep = jax.random.bernoulli(jax.random.key(42), 0.9,
                                (_B, _AD)).astype(jnp.float32)
    u = jax.random.uniform(jax.random.key(7), (_B, _R), dtype=jnp.float32)
    gumbel = -jnp.log(-jnp.log(u + 1e-8) + 1e-8)
    train_arr = jnp.asarray(training, jnp.float32).reshape(1, 1)

    hs2 = hidden_states.reshape(_B * _S, _H)
    tiny = lambda r, c: pl.BlockSpec((r, c), lambda i: (0, 0))
    return pl.pallas_call(
        _fused_kernel,
        grid=(_N_STEPS,),
        in_specs=[
            pl.BlockSpec((_RC, _H), lambda i: (i, 0)),
            tiny(_H, _AD),
            tiny(1, _AD),
            tiny(1, _AD),
            tiny(1, _AD),
            tiny(_AD, _R),
            tiny(1, _R),
            tiny(1, _R),
            tiny(_B, _AD),
            tiny(_B, _R),
            tiny(1, 1),
        ],
        out_specs=pl.BlockSpec((_B, _R), lambda i: (0, 0)),
        out_shape=jax.ShapeDtypeStruct((_B, _R), jnp.float32),
        scratch_shapes=[pltpu.VMEM((_B, _H), jnp.float32)],
    )(hs2, W1, b1.reshape(1, _AD), gamma.reshape(1, _AD),
      beta.reshape(1, _AD), W2, b2.reshape(1, _R),
      mask_logits.reshape(1, _R), keep, gumbel, train_arr)


# speedup vs baseline: 1.2357x; 1.0037x over previous
import functools

import jax
import jax.numpy as jnp
from jax import lax
from jax.experimental import pallas as pl
from jax.experimental.pallas import tpu as pltpu

_B, _S, _H = 4, 8192, 2048
_AD, _R, _K = 32, 16, 8
_TEMP = 0.1

_RC = 1024
_CPB = _S // _RC
_N_STEPS = (_B * _S) // _RC


def _tail(pooled, W1, b1, gamma, beta, W2, b2, mask_logits, keep, gumbel,
          training):
    x = jnp.dot(pooled, W1, preferred_element_type=jnp.float32) + b1
    mu = jnp.mean(x, axis=-1, keepdims=True)
    var = jnp.mean((x - mu) ** 2, axis=-1, keepdims=True)
    x = (x - mu) / jnp.sqrt(var + 1e-5) * gamma + beta
    x = 0.5 * x * (1.0 + lax.erf(x / jnp.sqrt(2.0).astype(jnp.float32)))
    x_dropped = jnp.where(keep > 0.5, x / 0.9, 0.0)
    is_training = training != 0
    x = jnp.where(is_training, x_dropped, x)
    logits = (jnp.dot(x, W2, preferred_element_type=jnp.float32) + b2
              + mask_logits)

    def khot(v):
        col = lax.broadcasted_iota(jnp.int32, v.shape, 1)
        cnt = jnp.zeros(v.shape, jnp.int32)
        for k in range(_R):
            vk = v[:, k][:, None]
            beats = (vk > v) | ((vk == v) & (k < col))
            cnt = cnt + beats.astype(jnp.int32)
        return (cnt < _K).astype(jnp.float32)

    noisy = (logits + gumbel) / _TEMP
    hard = khot(noisy)
    z = logits / _TEMP
    z = z - jnp.max(z, axis=-1, keepdims=True)
    ez = jnp.exp(z)
    soft = ez / jnp.sum(ez, axis=-1, keepdims=True)
    mask_train = hard + soft - soft
    mask_eval = khot(logits)
    return jnp.where(is_training, mask_train, mask_eval)


def _fused_kernel(hs_ref, W1_ref, b1_ref, gamma_ref, beta_ref, W2_ref,
                  b2_ref, ml_ref, keep_ref, gumbel_ref, train_ref,
                  out_ref, acc_ref):
    c = pl.program_id(0)
    bb = c // _CPB
    part = jnp.sum(hs_ref[...], axis=0, keepdims=True)

    @pl.when(c % _CPB == 0)
    def _():
        acc_ref[pl.ds(bb, 1), :] = part

    @pl.when(c % _CPB != 0)
    def _():
        acc_ref[pl.ds(bb, 1), :] += part

    @pl.when(c == _N_STEPS - 1)
    def _():
        pooled = acc_ref[...] * (1.0 / _S)
        out_ref[...] = _tail(
            pooled, W1_ref[...], b1_ref[...], gamma_ref[...], beta_ref[...],
            W2_ref[...], b2_ref[...], ml_ref[...], keep_ref[...],
            gumbel_ref[...], train_ref[0, 0])


def kernel(hidden_states, W1, b1, gamma, beta, W2, b2, mask_logits,
           training):
    keep = jax.random.bernoulli(jax.random.key(42), 0.9,
                                (_B, _AD)).astype(jnp.float32)
    u = jax.random.uniform(jax.random.key(7), (_B, _R), dtype=jnp.float32)
    gumbel = -jnp.log(-jnp.log(u + 1e-8) + 1e-8)
    train_arr = jnp.asarray(training, jnp.float32).reshape(1, 1)

    hs2 = hidden_states.reshape(_B * _S, _H)
    tiny = lambda r, c: pl.BlockSpec((r, c), lambda i: (0, 0))
    return pl.pallas_call(
        _fused_kernel,
        grid=(_N_STEPS,),
        in_specs=[
            pl.BlockSpec((_RC, _H), lambda i: (i, 0)),
            tiny(_H, _AD),
            tiny(1, _AD),
            tiny(1, _AD),
            tiny(1, _AD),
            tiny(_AD, _R),
            tiny(1, _R),
            tiny(1, _R),
            tiny(_B, _AD),
            tiny(_B, _R),
            tiny(1, 1),
        ],
        out_specs=pl.BlockSpec((_B, _R), lambda i: (0, 0)),
        out_shape=jax.ShapeDtypeStruct((_B, _R), jnp.float32),
        scratch_shapes=[pltpu.VMEM((_B, _H), jnp.float32)],
    )(hs2, W1, b1.reshape(1, _AD), gamma.reshape(1, _AD),
      beta.reshape(1, _AD), W2, b2.reshape(1, _R),
      mask_logits.reshape(1, _R), keep, gumbel, train_arr)
